# baseline (speedup 1.0000x reference)
import functools

import jax
import jax.numpy as jnp
from jax.experimental import pallas as pl
from jax.experimental.pallas import tpu as pltpu

E = 8
R = 64
TN = 256


def _fused_body(x_ref, xh_ref, gw_ref, gb_ref, u1_ref, v1_ref, b1_ref, u2_ref,
                v2_ref, b2_ref, out_ref):
    x = x_ref[...]
    logits = jnp.dot(x, gw_ref[...], preferred_element_type=jnp.float32)
    logits = logits + gb_ref[...]
    m = jnp.max(logits, axis=-1, keepdims=True)
    p = jnp.exp(logits - m)
    p = p / jnp.sum(p, axis=-1, keepdims=True)
    col = jax.lax.broadcasted_iota(jnp.int32, p.shape, 1)
    m1 = jnp.max(p, axis=-1, keepdims=True)
    i1 = jnp.min(jnp.where(p == m1, col, E), axis=-1, keepdims=True)
    pm = jnp.where(col == i1, -jnp.inf, p)
    m2 = jnp.max(pm, axis=-1, keepdims=True)
    i2 = jnp.min(jnp.where(pm == m2, col, E), axis=-1, keepdims=True)
    w = jnp.where(col == i1, m1, 0.0) + jnp.where(col == i2, m2, 0.0)
    w = w / (m1 + m2)

    xh = xh_ref[...]
    h1_all = jnp.dot(xh, u1_ref[...], preferred_element_type=jnp.float32)
    acc = jnp.zeros(out_ref.shape, jnp.float32)
    for e in range(E):
        h1 = h1_all[:, e * R:(e + 1) * R]
        h = jnp.dot(h1, v1_ref[e], preferred_element_type=jnp.float32)
        h = jax.nn.relu(h + b1_ref[e][None, :])
        h2 = jnp.dot(h, u2_ref[e], preferred_element_type=jnp.float32)
        o = jnp.dot(h2, v2_ref[e], preferred_element_type=jnp.float32)
        o = o + b2_ref[e][None, :]
        acc = acc + o * w[:, e:e + 1]
    out_ref[...] = acc


def kernel(x, gate_w, gate_b, u1, v1, b1, u2, v2, b2):
    n, d_in = x.shape
    d_h = v1.shape[-1]
    d_out = v2.shape[-1]
    f32 = jnp.float32
    xh = x.astype(jnp.float16).astype(f32)
    u1_all = u1.astype(f32).transpose(1, 0, 2).reshape(d_in, E * R)
    v1f = v1.astype(f32)
    b1f = b1.astype(f32)
    u2f = u2.astype(f32)
    v2f = v2.astype(f32)
    b2f = b2.astype(f32)
    gb2 = gate_b.reshape(1, E)
    grid = (n // TN,)
    out = pl.pallas_call(
        _fused_body,
        grid=grid,
        in_specs=[
            pl.BlockSpec((TN, d_in), lambda i: (i, 0)),
            pl.BlockSpec((TN, d_in), lambda i: (i, 0)),
            pl.BlockSpec((d_in, E), lambda i: (0, 0)),
            pl.BlockSpec((1, E), lambda i: (0, 0)),
            pl.BlockSpec((d_in, E * R), lambda i: (0, 0)),
            pl.BlockSpec((E, R, d_h), lambda i: (0, 0, 0)),
            pl.BlockSpec((E, d_h), lambda i: (0, 0)),
            pl.BlockSpec((E, d_h, R), lambda i: (0, 0, 0)),
            pl.BlockSpec((E, R, d_out), lambda i: (0, 0, 0)),
            pl.BlockSpec((E, d_out), lambda i: (0, 0)),
        ],
        out_specs=pl.BlockSpec((TN, d_out), lambda i: (i, 0)),
        out_shape=jax.ShapeDtypeStruct((n, d_out), jnp.float32),
        compiler_params=pltpu.CompilerParams(
            dimension_semantics=("arbitrary",),
        ),
    )(x, xh, gate_w, gb2, u1_all, v1f, b1f, u2f, v2f, b2f)
    return out


# speedup vs baseline: 1.3040x; 1.3040x over previous
import functools

import jax
import jax.numpy as jnp
from jax.experimental import pallas as pl
from jax.experimental.pallas import tpu as pltpu

E = 8
R = 64
TN = 256


def _fused_body(x_ref, xh_ref, gw_ref, gb_ref, u1_ref, v1_ref, b1_ref, u2_ref,
                v2_ref, b2_ref, out_ref):
    x = x_ref[...]
    logits = jnp.dot(x, gw_ref[...], preferred_element_type=jnp.float32)
    logits = logits + gb_ref[...]
    m = jnp.max(logits, axis=-1, keepdims=True)
    p = jnp.exp(logits - m)
    p = p / jnp.sum(p, axis=-1, keepdims=True)
    col = jax.lax.broadcasted_iota(jnp.int32, p.shape, 1)
    m1 = jnp.max(p, axis=-1, keepdims=True)
    i1 = jnp.min(jnp.where(p == m1, col, E), axis=-1, keepdims=True)
    pm = jnp.where(col == i1, -jnp.inf, p)
    m2 = jnp.max(pm, axis=-1, keepdims=True)
    i2 = jnp.min(jnp.where(pm == m2, col, E), axis=-1, keepdims=True)
    w = jnp.where(col == i1, m1, 0.0) + jnp.where(col == i2, m2, 0.0)
    w = w / (m1 + m2)

    xh = xh_ref[...]
    bf16 = jnp.bfloat16
    h1_all = jnp.dot(xh, u1_ref[...],
                     preferred_element_type=jnp.float32).astype(bf16)
    acc = jnp.zeros(out_ref.shape, jnp.float32)
    for e in range(E):
        h1 = h1_all[:, e * R:(e + 1) * R]
        h = jnp.dot(h1, v1_ref[e], preferred_element_type=jnp.float32)
        h = jax.nn.relu(h + b1_ref[e][None, :]).astype(bf16)
        h2 = jnp.dot(h, u2_ref[e],
                     preferred_element_type=jnp.float32).astype(bf16)
        o = jnp.dot(h2, v2_ref[e], preferred_element_type=jnp.float32)
        o = o + b2_ref[e][None, :]
        acc = acc + o * w[:, e:e + 1]
    out_ref[...] = acc


def kernel(x, gate_w, gate_b, u1, v1, b1, u2, v2, b2):
    n, d_in = x.shape
    d_h = v1.shape[-1]
    d_out = v2.shape[-1]
    f32 = jnp.float32
    bf16 = jnp.bfloat16
    xh = x.astype(jnp.float16).astype(bf16)
    u1_all = u1.astype(bf16).transpose(1, 0, 2).reshape(d_in, E * R)
    v1f = v1.astype(bf16)
    b1f = b1.astype(f32)
    u2f = u2.astype(bf16)
    v2f = v2.astype(bf16)
    b2f = b2.astype(f32)
    gb2 = gate_b.reshape(1, E)
    grid = (n // TN,)
    out = pl.pallas_call(
        _fused_body,
        grid=grid,
        in_specs=[
            pl.BlockSpec((TN, d_in), lambda i: (i, 0)),
            pl.BlockSpec((TN, d_in), lambda i: (i, 0)),
            pl.BlockSpec((d_in, E), lambda i: (0, 0)),
            pl.BlockSpec((1, E), lambda i: (0, 0)),
            pl.BlockSpec((d_in, E * R), lambda i: (0, 0)),
            pl.BlockSpec((E, R, d_h), lambda i: (0, 0, 0)),
            pl.BlockSpec((E, d_h), lambda i: (0, 0)),
            pl.BlockSpec((E, d_h, R), lambda i: (0, 0, 0)),
            pl.BlockSpec((E, R, d_out), lambda i: (0, 0, 0)),
            pl.BlockSpec((E, d_out), lambda i: (0, 0)),
        ],
        out_specs=pl.BlockSpec((TN, d_out), lambda i: (i, 0)),
        out_shape=jax.ShapeDtypeStruct((n, d_out), jnp.float32),
        compiler_params=pltpu.CompilerParams(
            dimension_semantics=("arbitrary",),
        ),
    )(x, xh, gate_w, gb2, u1_all, v1f, b1f, u2f, v2f, b2f)
    return out
